# baseline (speedup 1.0000x reference)
import jax
import jax.numpy as jnp
from jax.experimental import pallas as pl
from jax.experimental.pallas import tpu as pltpu

_GRID = 4


def _copy_body(pe_ref, o_ref):
    o_ref[...] = pe_ref[...]


def kernel(idx, pe):
    T = idx.shape[1]
    D = pe.shape[1]
    rows = T // _GRID
    out = pl.pallas_call(
        _copy_body,
        grid=(_GRID,),
        out_shape=jax.ShapeDtypeStruct((T, D), pe.dtype),
        in_specs=[pl.BlockSpec((rows, D), lambda i: (i, 0))],
        out_specs=pl.BlockSpec((rows, D), lambda i: (i, 0)),
    )(pe)
    return out[None, :, :]


# speedup vs baseline: 1.7164x; 1.7164x over previous
import jax
import jax.numpy as jnp
from jax.experimental import pallas as pl
from jax.experimental.pallas import tpu as pltpu

_CHUNKS = 4


def _copy_body(pe_hbm, o_hbm, scratch, in_sems, out_sems):
    T = scratch.shape[0]
    rows = T // _CHUNKS
    for k in range(_CHUNKS):
        sl = pl.ds(k * rows, rows)
        pltpu.make_async_copy(
            pe_hbm.at[sl, :], scratch.at[sl, :], in_sems.at[k]
        ).start()
    for k in range(_CHUNKS):
        sl = pl.ds(k * rows, rows)
        pltpu.make_async_copy(
            pe_hbm.at[sl, :], scratch.at[sl, :], in_sems.at[k]
        ).wait()
        pltpu.make_async_copy(
            scratch.at[sl, :], o_hbm.at[sl, :], out_sems.at[k]
        ).start()
    for k in range(_CHUNKS):
        sl = pl.ds(k * rows, rows)
        pltpu.make_async_copy(
            scratch.at[sl, :], o_hbm.at[sl, :], out_sems.at[k]
        ).wait()


def kernel(idx, pe):
    T = idx.shape[1]
    D = pe.shape[1]
    out = pl.pallas_call(
        _copy_body,
        out_shape=jax.ShapeDtypeStruct((T, D), pe.dtype),
        in_specs=[pl.BlockSpec(memory_space=pl.ANY)],
        out_specs=pl.BlockSpec(memory_space=pl.ANY),
        scratch_shapes=[
            pltpu.VMEM((T, D), pe.dtype),
            pltpu.SemaphoreType.DMA((_CHUNKS,)),
            pltpu.SemaphoreType.DMA((_CHUNKS,)),
        ],
    )(pe)
    return out[None, :, :]
